# baseline (speedup 1.0000x reference)
import functools

import jax
import jax.numpy as jnp
from jax import lax
from jax.experimental import pallas as pl
from jax.experimental.pallas import tpu as pltpu
from jax.experimental.pallas import tpu_sc as plsc

N_PTS = 150000
NROWS = 1184
R0 = 37
R1 = 37
CAN = 512 * 512
NQ = 4096
NV = 64
HIST_W = NQ * NV + NV
CHUNK_CELLS = CAN // 16
CHUNK_HIST = NQ * NV // 16


def _sc_kernel(cells_hbm, valid_hbm, cluster_hbm, radar_hbm, out_hbm,
               inv_s, hist_s, res_s,
               radar_v, zbuf_v, zbuf2_v, jbuf_v, valid_v, vals_v, slot_v,
               cells_v, bins_v, ones_v, resc_v, qslot_v, qval_v,
               qslot2_v, qval2_v, hist_t_v, sem, sem2, sem3):
    c = lax.axis_index("c")
    s = lax.axis_index("s")
    w = s * 2 + c
    lane = lax.iota(jnp.int32, 16)
    off = pl.multiple_of(w * (R0 * 128), 8)

    cp_r = pltpu.async_copy(radar_hbm, radar_v, sem3)
    cp_c = pltpu.async_copy(cells_hbm.at[pl.ds(off, R0 * 128)],
                            cells_v, sem3)
    cp_v = pltpu.async_copy(valid_hbm.at[pl.ds(off, R0 * 128)],
                            valid_v, sem3)

    neg1 = jnp.full((16,), -1, jnp.int32)
    zero = jnp.zeros((16,), jnp.int32)
    one = jnp.full((16,), 1, jnp.int32)

    def _fill(i, _):
        b = i * 128
        for k in range(8):
            zbuf_v[pl.ds(b + k * 16, 16)] = neg1
        return 0
    lax.fori_loop(0, 8, _fill, 0)

    def _icpy(i, _):
        pltpu.async_copy(zbuf_v, inv_s.at[pl.ds(s * CHUNK_CELLS + i * 1024,
                                                1024)], sem)
        return 0
    lax.fori_loop(0, 16, _icpy, 0)

    for k in range(8):
        ones_v[pl.ds(k * 16, 16)] = one

    def _jfill(i, _):
        jbuf_v[pl.ds(i * 16, 16)] = lane + (s * 256 + i * 16)
        return 0
    lax.fori_loop(0, 16, _jfill, 0)

    def _zfill(i, _):
        b = i * 128
        for k in range(8):
            zbuf2_v[pl.ds(b + k * 16, 16)] = zero
        return 0
    lax.fori_loop(0, 8, _zfill, 0)

    def _zcpy(i, _):
        pltpu.async_copy(zbuf2_v, hist_s.at[pl.ds(s * CHUNK_HIST + i * 1024,
                                                  1024)], sem)
        return 0
    lax.fori_loop(0, 16, _zcpy, 0)

    cp_r.wait()
    cp_c.wait()
    cp_v.wait()

    def _gfire(j, _):
        pltpu.async_copy(cluster_hbm.at[valid_v.at[pl.ds(j * 128, 128)]],
                         vals_v.at[pl.ds(j * 128, 128)], sem2)
        return 0
    lax.fori_loop(0, R0, _gfire, 0)

    def _idrain(i, _):
        pltpu.make_async_copy(
            zbuf_v, inv_s.at[pl.ds(s * CHUNK_CELLS + i * 1024, 1024)],
            sem).wait()
        pltpu.make_async_copy(
            zbuf2_v, hist_s.at[pl.ds(s * CHUNK_HIST + i * 1024, 1024)],
            sem).wait()
        return 0
    lax.fori_loop(0, 16, _idrain, 0)

    @pl.when(s == 0)
    def _():
        pltpu.sync_copy(zbuf_v.at[pl.ds(0, 16)], inv_s.at[pl.ds(CAN, 16)])
        pltpu.sync_copy(zbuf2_v.at[pl.ds(0, NV)],
                        hist_s.at[pl.ds(NQ * NV, NV)])

    plsc.subcore_barrier()

    pltpu.sync_copy(jbuf_v.at[pl.ds(0, 128)],
                    inv_s.at[radar_v.at[2 * s]])
    pltpu.sync_copy(jbuf_v.at[pl.ds(128, 128)],
                    inv_s.at[radar_v.at[2 * s + 1]])
    plsc.subcore_barrier()


    def _sfire(j, _):
        pltpu.async_copy(inv_s.at[cells_v.at[pl.ds(j * 128, 128)]],
                         slot_v.at[pl.ds(j * 128, 128)], sem)
        return 0
    lax.fori_loop(0, R0, _sfire, 0)

    pltpu.make_async_copy(cells_hbm.at[pl.ds(0, R0 * 128)], slot_v,
                          sem).wait()
    pltpu.make_async_copy(cells_hbm.at[pl.ds(0, R0 * 128)], vals_v,
                          sem2).wait()

    trash0 = lane + NQ * NV

    def _bins(h, _):
        for u in range(2):
            i = h * 2 + u
            sl = slot_v[pl.ds(i * 16, 16)]
            vv = vals_v[pl.ds(i * 16, 16)]
            binv = jnp.where(sl >= 0, sl * NV + vv, trash0 + (i % 4) * 16)
            bins_v[i // 8, pl.ds((i % 8) * 16, 16)] = binv
        return 0
    lax.fori_loop(0, R0 * 4, _bins, 0)

    def _afire(j, _):
        pltpu.async_copy(ones_v, hist_s.at[bins_v.at[j]], sem, add=True)
        return 0
    lax.fori_loop(0, R0, _afire, 0)

    pltpu.make_async_copy(cells_hbm.at[pl.ds(0, R0 * 128)], cells_v,
                          sem).wait()
    plsc.subcore_barrier()

    pltpu.async_copy(hist_s.at[pl.ds(s * CHUNK_HIST, 4096)],
                     hist_t_v.at[pl.ds(0, 4096)], sem)

    def _p3piece(p, _):
        buf = (p % 2) * 4096
        pltpu.make_async_copy(hist_s.at[pl.ds(0, 4096)],
                              hist_t_v.at[pl.ds(buf, 4096)], sem).wait()

        @pl.when(p < 3)
        def _():
            pltpu.async_copy(
                hist_s.at[pl.ds(s * CHUNK_HIST + (p + 1) * 4096, 4096)],
                hist_t_v.at[pl.ds(4096 - buf, 4096)], sem)

        def _p3g(g, _):
            base_idx = lane * NV + (g * (16 * NV) + buf)

            def _p3b(b, curs):
                c0, c1, c2, c3 = curs
                h0 = plsc.load_gather(hist_t_v, [base_idx + b])
                h1 = plsc.load_gather(hist_t_v, [base_idx + (b + 16)])
                h2 = plsc.load_gather(hist_t_v, [base_idx + (b + 32)])
                h3 = plsc.load_gather(hist_t_v, [base_idx + (b + 48)])
                return (jnp.where(h0 > 0, b, c0),
                        jnp.where(h1 > 0, b + 16, c1),
                        jnp.where(h2 > 0, b + 32, c2),
                        jnp.where(h3 > 0, b + 48, c3))
            m1 = jnp.full((16,), -1, jnp.int32)
            c0, c1, c2, c3 = lax.fori_loop(0, 16, _p3b, (m1, m1, m1, m1))
            cur = jnp.maximum(jnp.maximum(c0, c1), jnp.maximum(c2, c3))
            resc_v[pl.ds(p * 64 + g * 16, 16)] = cur
            return 0
        lax.fori_loop(0, 4, _p3g, 0)
        return 0

    lax.fori_loop(0, 4, _p3piece, 0)
    pltpu.sync_copy(resc_v, res_s.at[pl.ds(s * 256, 256)])
    plsc.subcore_barrier()

    cq0 = pltpu.async_copy(inv_s.at[radar_v.at[2 * s]], qslot_v, sem)
    cq1 = pltpu.async_copy(inv_s.at[radar_v.at[2 * s + 1]], qslot2_v, sem2)
    cq0.wait()
    cr0 = pltpu.async_copy(res_s.at[qslot_v], qval_v, sem)
    cq1.wait()
    cr1 = pltpu.async_copy(res_s.at[qslot2_v], qval2_v, sem2)
    cr0.wait()
    co0 = pltpu.async_copy(qval_v, out_hbm.at[c, pl.ds(s * 256, 128)], sem)
    cr1.wait()
    co1 = pltpu.async_copy(qval2_v, out_hbm.at[c, pl.ds(s * 256 + 128, 128)],
                           sem2)
    co0.wait()
    co1.wait()


_mesh = plsc.VectorSubcoreMesh(core_axis_name="c", subcore_axis_name="s")

_call = functools.partial(
    pl.kernel,
    mesh=_mesh,
    out_type=jax.ShapeDtypeStruct((2, NQ), jnp.int32),
    compiler_params=pltpu.CompilerParams(needs_layout_passes=False),
    scratch_types=[
        pltpu.VMEM_SHARED((CAN + 16,), jnp.int32),
        pltpu.VMEM_SHARED((HIST_W,), jnp.int32),
        pltpu.VMEM_SHARED((NQ,), jnp.int32),
        pltpu.VMEM((32, 128), jnp.int32),
        pltpu.VMEM((1024,), jnp.int32),
        pltpu.VMEM((1024,), jnp.int32),
        pltpu.VMEM((256,), jnp.int32),
        pltpu.VMEM((R0 * 128,), jnp.int32),
        pltpu.VMEM((R0 * 128,), jnp.int32),
        pltpu.VMEM((R0 * 128,), jnp.int32),
        pltpu.VMEM((R0 * 128,), jnp.int32),
        pltpu.VMEM((R0, 128), jnp.int32),
        pltpu.VMEM((128,), jnp.int32),
        pltpu.VMEM((256,), jnp.int32),
        pltpu.VMEM((128,), jnp.int32),
        pltpu.VMEM((128,), jnp.int32),
        pltpu.VMEM((128,), jnp.int32),
        pltpu.VMEM((128,), jnp.int32),
        pltpu.VMEM((8192,), jnp.int32),
        pltpu.SemaphoreType.DMA,
        pltpu.SemaphoreType.DMA,
        pltpu.SemaphoreType.DMA,
    ],
)(_sc_kernel)


def kernel(li_coor, li_valid, lidar_cluster, radar_ind):
    pad = NROWS * 128 - N_PTS
    cell = li_coor[:, 1].astype(jnp.int32) * 512 + li_coor[:, 2].astype(
        jnp.int32)
    cells3 = jnp.pad(cell, (0, pad), constant_values=CAN)
    valid3 = jnp.pad(li_valid.astype(jnp.int32), (0, pad))
    cluster_i32 = lidar_cluster.astype(jnp.int32)
    radar2 = radar_ind.astype(jnp.int32).reshape(32, 128)
    part = _call(cells3, valid3, cluster_i32, radar2)
    m = jnp.maximum(part[0], part[1])
    return jnp.where(m >= 0, m, 0).astype(jnp.int16)


# speedup vs baseline: 1.0299x; 1.0299x over previous
import functools

import jax
import jax.numpy as jnp
from jax import lax
from jax.experimental import pallas as pl
from jax.experimental.pallas import tpu as pltpu
from jax.experimental.pallas import tpu_sc as plsc

N_PTS = 150000
NROWS = 1184
R0 = 37
R1 = 37
CAN = 512 * 512
NQ = 4096
NV = 64
HIST_W = NQ * NV + NV
CHUNK_CELLS = CAN // 16
CHUNK_HIST = NQ * NV // 16


def _sc_kernel(cells_hbm, valid_hbm, cluster_hbm, radar_hbm, out_hbm,
               inv_s, hist_s, res_s,
               radar_v, zbuf_v, zbuf2_v, jbuf_v, valid_v, vals_v, slot_v,
               cells_v, bins_v, ones_v, resc_v, qslot_v, qval_v,
               qslot2_v, qval2_v, hist_t_v, sem, sem2, sem3):
    c = lax.axis_index("c")
    s = lax.axis_index("s")
    w = s * 2 + c
    lane = lax.iota(jnp.int32, 16)
    cp_r = pltpu.async_copy(radar_hbm, radar_v, sem3)
    cp_c = pltpu.async_copy(cells_hbm.at[w], cells_v, sem3)
    cp_v = pltpu.async_copy(valid_hbm.at[w], valid_v, sem3)

    neg1 = jnp.full((16,), -1, jnp.int32)
    zero = jnp.zeros((16,), jnp.int32)
    one = jnp.full((16,), 1, jnp.int32)

    def _fill(i, _):
        b = i * 128
        for k in range(8):
            zbuf_v[pl.ds(b + k * 16, 16)] = neg1
        return 0
    lax.fori_loop(0, 8, _fill, 0)

    def _icpy(i, _):
        pltpu.async_copy(zbuf_v, inv_s.at[pl.ds(s * CHUNK_CELLS + i * 1024,
                                                1024)], sem)
        return 0
    lax.fori_loop(0, 16, _icpy, 0)

    for k in range(8):
        ones_v[pl.ds(k * 16, 16)] = one

    def _jfill(i, _):
        jbuf_v[pl.ds(i * 16, 16)] = lane + (s * 256 + i * 16)
        return 0
    lax.fori_loop(0, 16, _jfill, 0)

    def _zfill(i, _):
        b = i * 128
        for k in range(8):
            zbuf2_v[pl.ds(b + k * 16, 16)] = zero
        return 0
    lax.fori_loop(0, 8, _zfill, 0)

    def _zcpy(i, _):
        pltpu.async_copy(zbuf2_v, hist_s.at[pl.ds(s * CHUNK_HIST + i * 1024,
                                                  1024)], sem)
        return 0
    lax.fori_loop(0, 16, _zcpy, 0)

    cp_r.wait()
    cp_c.wait()
    cp_v.wait()

    def _gfire(j, _):
        pltpu.async_copy(cluster_hbm.at[valid_v.at[j]], vals_v.at[j], sem2)
        return 0
    lax.fori_loop(0, R0, _gfire, 0)

    def _idrain(i, _):
        pltpu.make_async_copy(
            zbuf_v, inv_s.at[pl.ds(s * CHUNK_CELLS + i * 1024, 1024)],
            sem).wait()
        pltpu.make_async_copy(
            zbuf2_v, hist_s.at[pl.ds(s * CHUNK_HIST + i * 1024, 1024)],
            sem).wait()
        return 0
    lax.fori_loop(0, 16, _idrain, 0)

    @pl.when(s == 0)
    def _():
        pltpu.sync_copy(zbuf_v.at[pl.ds(0, 16)], inv_s.at[pl.ds(CAN, 16)])
        pltpu.sync_copy(zbuf2_v.at[pl.ds(0, NV)],
                        hist_s.at[pl.ds(NQ * NV, NV)])

    plsc.subcore_barrier()

    pltpu.sync_copy(jbuf_v.at[pl.ds(0, 128)],
                    inv_s.at[radar_v.at[2 * s]])
    pltpu.sync_copy(jbuf_v.at[pl.ds(128, 128)],
                    inv_s.at[radar_v.at[2 * s + 1]])
    plsc.subcore_barrier()


    def _sfire(j, _):
        pltpu.async_copy(inv_s.at[cells_v.at[j]], slot_v.at[j], sem)
        return 0
    lax.fori_loop(0, R0, _sfire, 0)

    pltpu.make_async_copy(valid_hbm.at[w], slot_v, sem).wait()
    pltpu.make_async_copy(valid_hbm.at[w], vals_v, sem2).wait()

    trash0 = lane + NQ * NV

    def _bins(h, _):
        for u in range(2):
            i = h * 2 + u
            j = i // 8
            col = (i % 8) * 16
            sl = slot_v[j, pl.ds(col, 16)]
            vv = vals_v[j, pl.ds(col, 16)]
            binv = jnp.where(sl >= 0, sl * NV + vv, trash0 + (i % 4) * 16)
            bins_v[j, pl.ds(col, 16)] = binv
        return 0
    lax.fori_loop(0, R0 * 4, _bins, 0)

    def _afire(j, _):
        pltpu.async_copy(ones_v, hist_s.at[bins_v.at[j]], sem, add=True)
        return 0
    lax.fori_loop(0, R0, _afire, 0)

    pltpu.make_async_copy(valid_hbm.at[w], bins_v, sem).wait()
    plsc.subcore_barrier()

    pltpu.async_copy(hist_s.at[pl.ds(s * CHUNK_HIST, 4096)],
                     hist_t_v.at[pl.ds(0, 4096)], sem)

    def _p3piece(p, _):
        buf = (p % 2) * 4096
        pltpu.make_async_copy(hist_s.at[pl.ds(0, 4096)],
                              hist_t_v.at[pl.ds(buf, 4096)], sem).wait()

        @pl.when(p < 3)
        def _():
            pltpu.async_copy(
                hist_s.at[pl.ds(s * CHUNK_HIST + (p + 1) * 4096, 4096)],
                hist_t_v.at[pl.ds(4096 - buf, 4096)], sem)

        def _p3g(g, _):
            base_idx = lane * NV + (g * (16 * NV) + buf)

            def _p3b(b, curs):
                c0, c1, c2, c3 = curs
                h0 = plsc.load_gather(hist_t_v, [base_idx + b])
                h1 = plsc.load_gather(hist_t_v, [base_idx + (b + 16)])
                h2 = plsc.load_gather(hist_t_v, [base_idx + (b + 32)])
                h3 = plsc.load_gather(hist_t_v, [base_idx + (b + 48)])
                return (jnp.where(h0 > 0, b, c0),
                        jnp.where(h1 > 0, b + 16, c1),
                        jnp.where(h2 > 0, b + 32, c2),
                        jnp.where(h3 > 0, b + 48, c3))
            m1 = jnp.full((16,), -1, jnp.int32)
            c0, c1, c2, c3 = lax.fori_loop(0, 16, _p3b, (m1, m1, m1, m1))
            cur = jnp.maximum(jnp.maximum(c0, c1), jnp.maximum(c2, c3))
            resc_v[pl.ds(p * 64 + g * 16, 16)] = cur
            return 0
        lax.fori_loop(0, 4, _p3g, 0)
        return 0

    lax.fori_loop(0, 4, _p3piece, 0)
    pltpu.sync_copy(resc_v, res_s.at[pl.ds(s * 256, 256)])
    plsc.subcore_barrier()

    cq0 = pltpu.async_copy(inv_s.at[radar_v.at[2 * s]], qslot_v, sem)
    cq1 = pltpu.async_copy(inv_s.at[radar_v.at[2 * s + 1]], qslot2_v, sem2)
    cq0.wait()
    cr0 = pltpu.async_copy(res_s.at[qslot_v], qval_v, sem)
    cq1.wait()
    cr1 = pltpu.async_copy(res_s.at[qslot2_v], qval2_v, sem2)
    cr0.wait()
    co0 = pltpu.async_copy(qval_v, out_hbm.at[c, pl.ds(s * 256, 128)], sem)
    cr1.wait()
    co1 = pltpu.async_copy(qval2_v, out_hbm.at[c, pl.ds(s * 256 + 128, 128)],
                           sem2)
    co0.wait()
    co1.wait()


_mesh = plsc.VectorSubcoreMesh(core_axis_name="c", subcore_axis_name="s")

_call = functools.partial(
    pl.kernel,
    mesh=_mesh,
    out_type=jax.ShapeDtypeStruct((2, NQ), jnp.int32),
    compiler_params=pltpu.CompilerParams(needs_layout_passes=False),
    scratch_types=[
        pltpu.VMEM_SHARED((CAN + 16,), jnp.int32),
        pltpu.VMEM_SHARED((HIST_W,), jnp.int32),
        pltpu.VMEM_SHARED((NQ,), jnp.int32),
        pltpu.VMEM((32, 128), jnp.int32),
        pltpu.VMEM((1024,), jnp.int32),
        pltpu.VMEM((1024,), jnp.int32),
        pltpu.VMEM((256,), jnp.int32),
        pltpu.VMEM((R0, 128), jnp.int32),
        pltpu.VMEM((R0, 128), jnp.int32),
        pltpu.VMEM((R0, 128), jnp.int32),
        pltpu.VMEM((R0, 128), jnp.int32),
        pltpu.VMEM((R0, 128), jnp.int32),
        pltpu.VMEM((128,), jnp.int32),
        pltpu.VMEM((256,), jnp.int32),
        pltpu.VMEM((128,), jnp.int32),
        pltpu.VMEM((128,), jnp.int32),
        pltpu.VMEM((128,), jnp.int32),
        pltpu.VMEM((128,), jnp.int32),
        pltpu.VMEM((8192,), jnp.int32),
        pltpu.SemaphoreType.DMA,
        pltpu.SemaphoreType.DMA,
        pltpu.SemaphoreType.DMA,
    ],
)(_sc_kernel)


def kernel(li_coor, li_valid, lidar_cluster, radar_ind):
    pad = NROWS * 128 - N_PTS
    cell = li_coor[:, 1].astype(jnp.int32) * 512 + li_coor[:, 2].astype(
        jnp.int32)
    cells3 = jnp.pad(cell, (0, pad), constant_values=CAN).reshape(
        32, R0, 128)
    valid3 = jnp.pad(li_valid.astype(jnp.int32), (0, pad)).reshape(
        32, R0, 128)
    cluster_i32 = lidar_cluster.astype(jnp.int32)
    radar2 = radar_ind.astype(jnp.int32).reshape(32, 128)
    part = _call(cells3, valid3, cluster_i32, radar2)
    m = jnp.maximum(part[0], part[1])
    return jnp.where(m >= 0, m, 0).astype(jnp.int16)
